# baseline (speedup 1.0000x reference)
import functools

import jax
import jax.numpy as jnp
from jax import lax
from jax.experimental import pallas as pl
from jax.experimental.pallas import tpu as pltpu
from jax.experimental.pallas import tpu_sc as plsc

_NUM_GRAPHS = 128
_D_IN = 256
_D_HID = 512
_D_OUT = 256
_N_NODES = 10000

_NC = 2
_NS = 16
_NW = _NC * _NS
_RPW = 160
_N_SC = _NW * _RPW
_N_TC = _N_NODES - _N_SC
_TC_BLK = 976
_CHUNK = 80
_NCHUNK = _RPW // _CHUNK
_GROUPS = _CHUNK // 16


def _make_seg_pool():
  mesh = plsc.VectorSubcoreMesh(core_axis_name="c", subcore_axis_name="s")

  @functools.partial(
      pl.kernel,
      out_type=jax.ShapeDtypeStruct((_NW, _NUM_GRAPHS, _D_IN), jnp.float32),
      mesh=mesh,
      scratch_types=[
          pltpu.VMEM((_RPW, _D_IN), jnp.float32),
          pltpu.VMEM((_RPW,), jnp.int32),
          pltpu.VMEM((_NUM_GRAPHS, _D_IN), jnp.float32),
          pltpu.SemaphoreType.DMA,
          pltpu.SemaphoreType.DMA,
      ],
  )
  def seg_pool(x_hbm, idx_hbm, out_hbm, xv, iv, acc, sem_a, sem_i):
    cid = lax.axis_index("c")
    sid = lax.axis_index("s")
    wid = cid * _NS + sid
    base = wid * _RPW

    def x_copy(q, sem):
      off = q * _CHUNK
      return pltpu.make_async_copy(
          x_hbm.at[pl.ds(base + off, _CHUNK), :],
          xv.at[pl.ds(off, _CHUNK), :], sem)

    def idx_copy():
      return pltpu.make_async_copy(idx_hbm.at[pl.ds(base, _RPW)],
                                   iv.at[pl.ds(0, _RPW)], sem_i)

    ncol = _D_IN // 16

    def group_body(t):
      gvec = iv[pl.ds(t * 16, 16)]
      g0 = gvec[0]
      g15 = gvec[15]

      @pl.when(g0 == g15)
      def _():
        for c in range(ncol):
          vs = [xv[t * 16 + j, pl.ds(c * 16, 16)] for j in range(16)]
          while len(vs) > 1:
            vs = [vs[2 * i] + vs[2 * i + 1] for i in range(len(vs) // 2)]
          plsc.addupdate(acc.at[g0, pl.ds(c * 16, 16)], vs[0])

      @pl.when(g0 != g15)
      def _():
        gs = [gvec[j] for j in range(16)]

        def load_row(j):
          r = t * 16 + j
          return [xv[r, pl.ds(c * 16, 16)] for c in range(ncol)]

        vs = load_row(0)
        for j in range(16):
          nxt = load_row(j + 1) if j < 15 else None
          for c in range(ncol):
            plsc.addupdate(acc.at[gs[j], pl.ds(c * 16, 16)], vs[c])
          vs = nxt

    for q in range(_NCHUNK):
      x_copy(q, sem_a).start()
    idx_copy().start()

    z = jnp.zeros((16,), jnp.float32)

    def zero_row(r, carry):
      for c in range(_D_IN // 16):
        acc[r, pl.ds(c * 16, 16)] = z
      return carry

    lax.fori_loop(0, _NUM_GRAPHS, zero_row, 0)

    idx_copy().wait()

    def chunk_body(q, carry):
      x_copy(q, sem_a).wait()
      plsc.parallel_loop(q * _GROUPS, (q + 1) * _GROUPS)(group_body)
      return carry

    lax.fori_loop(0, _NCHUNK, chunk_body, 0)

    pltpu.sync_copy(acc, out_hbm.at[wid])

  return seg_pool


_seg_pool = _make_seg_pool()


def _tc_pool_body(x_ref, idx_ref, o_ref):
  ids = idx_ref[...]
  cols = lax.broadcasted_iota(jnp.int32, (_N_TC, _NUM_GRAPHS), 1)
  onehot = (cols == ids).astype(jnp.float32)
  o_ref[...] = lax.dot_general(
      onehot, x_ref[...], (((0,), (0,)), ((), ())),
      preferred_element_type=jnp.float32)


def _tc_pool(x_tail, idx_tail):
  return pl.pallas_call(
      _tc_pool_body,
      out_shape=jax.ShapeDtypeStruct((_NUM_GRAPHS, _D_IN), jnp.float32),
  )(x_tail, idx_tail.reshape(_N_TC, 1))


def _mlp_body(p_ref, t_ref, w1_ref, b1_ref, w2_ref, b2_ref, o_ref):
  pooled = jnp.sum(p_ref[...], axis=0) + t_ref[...]
  h = jnp.dot(pooled, w1_ref[...], preferred_element_type=jnp.float32)
  h = h + b1_ref[...]
  h = jnp.where(h >= 0.0, h, 0.01 * h)
  o_ref[...] = (
      jnp.dot(h, w2_ref[...], preferred_element_type=jnp.float32)
      + b2_ref[...]
  )


def _mlp(partials, tc_part, W1, b1, W2, b2):
  return pl.pallas_call(
      _mlp_body,
      out_shape=jax.ShapeDtypeStruct((_NUM_GRAPHS, _D_OUT), jnp.float32),
  )(partials, tc_part, W1, b1.reshape(1, _D_HID), W2, b2.reshape(1, _D_OUT))


def kernel(x, edge_index, batch, W1, b1, W2, b2):
  del edge_index
  idx = batch.astype(jnp.int32)
  tc_part = _tc_pool(x[_N_SC:], idx[_N_SC:])
  partials = _seg_pool(x, idx)
  return _mlp(partials, tc_part, W1, b1, W2, b2)


# speedup vs baseline: 1.0448x; 1.0448x over previous
import functools

import jax
import jax.numpy as jnp
from jax import lax
from jax.experimental import pallas as pl
from jax.experimental.pallas import tpu as pltpu
from jax.experimental.pallas import tpu_sc as plsc

_NUM_GRAPHS = 128
_D_IN = 256
_D_HID = 512
_D_OUT = 256
_N_NODES = 10000

_NC = 2
_NS = 16
_NW = _NC * _NS
_RPW = 160
_N_SC = _NW * _RPW
_N_TC = _N_NODES - _N_SC
_TC_BLK = 976
_CHUNK = 80
_NCHUNK = _RPW // _CHUNK
_GROUPS = _CHUNK // 16


def _make_seg_pool():
  mesh = plsc.VectorSubcoreMesh(core_axis_name="c", subcore_axis_name="s")

  @functools.partial(
      pl.kernel,
      out_type=jax.ShapeDtypeStruct((_NW, _NUM_GRAPHS, _D_IN), jnp.float32),
      mesh=mesh,
      scratch_types=[
          pltpu.VMEM((_RPW, _D_IN), jnp.float32),
          pltpu.VMEM((_RPW,), jnp.int32),
          pltpu.VMEM((_NUM_GRAPHS, _D_IN), jnp.float32),
          pltpu.SemaphoreType.DMA,
          pltpu.SemaphoreType.DMA,
      ],
  )
  def seg_pool(x_hbm, idx_hbm, out_hbm, xv, iv, acc, sem_a, sem_i):
    cid = lax.axis_index("c")
    sid = lax.axis_index("s")
    wid = cid * _NS + sid
    base = wid * _RPW

    def x_copy(q, sem):
      off = q * _CHUNK
      return pltpu.make_async_copy(
          x_hbm.at[pl.ds(base + off, _CHUNK), :],
          xv.at[pl.ds(off, _CHUNK), :], sem)

    def idx_copy():
      return pltpu.make_async_copy(idx_hbm.at[pl.ds(base, _RPW)],
                                   iv.at[pl.ds(0, _RPW)], sem_i)

    ncol = _D_IN // 16

    def group_body(t):
      gvec = iv[pl.ds(t * 16, 16)]
      g0 = gvec[0]
      g15 = gvec[15]

      @pl.when(g0 == g15)
      def _():
        for c in range(ncol):
          vs = [xv[t * 16 + j, pl.ds(c * 16, 16)] for j in range(16)]
          while len(vs) > 1:
            vs = [vs[2 * i] + vs[2 * i + 1] for i in range(len(vs) // 2)]
          plsc.addupdate(acc.at[g0, pl.ds(c * 16, 16)], vs[0])

      @pl.when(g0 != g15)
      def _():
        gs = [gvec[j] for j in range(16)]

        def load_row(j):
          r = t * 16 + j
          return [xv[r, pl.ds(c * 16, 16)] for c in range(ncol)]

        vs = load_row(0)
        for j in range(16):
          nxt = load_row(j + 1) if j < 15 else None
          for c in range(ncol):
            plsc.addupdate(acc.at[gs[j], pl.ds(c * 16, 16)], vs[c])
          vs = nxt

    for q in range(_NCHUNK):
      x_copy(q, sem_a).start()
    idx_copy().start()

    z = jnp.zeros((16,), jnp.float32)

    def zero_row(r, carry):
      for c in range(_D_IN // 16):
        acc[r, pl.ds(c * 16, 16)] = z
      return carry

    lax.fori_loop(0, _NUM_GRAPHS, zero_row, 0)

    idx_copy().wait()

    def chunk_body(q, carry):
      x_copy(q, sem_a).wait()
      plsc.parallel_loop(q * _GROUPS, (q + 1) * _GROUPS)(group_body)
      return carry

    lax.fori_loop(0, _NCHUNK, chunk_body, 0)

    pltpu.sync_copy(acc, out_hbm.at[wid])

  return seg_pool


_seg_pool = _make_seg_pool()


def _tc_pool_body(x_ref, idx_ref, o_ref):
  ids = idx_ref[...]
  rows = lax.broadcasted_iota(jnp.int32, (_NUM_GRAPHS, _N_TC), 0)
  onehot = (rows == ids).astype(jnp.float32)
  o_ref[...] = jnp.dot(onehot, x_ref[...],
                       preferred_element_type=jnp.float32)


def _tc_pool(x_tail, idx_tail):
  return pl.pallas_call(
      _tc_pool_body,
      out_shape=jax.ShapeDtypeStruct((_NUM_GRAPHS, _D_IN), jnp.float32),
  )(x_tail, idx_tail.reshape(1, _N_TC))


def _mlp_body(p_ref, t_ref, w1_ref, b1_ref, w2_ref, b2_ref, o_ref):
  pooled = jnp.sum(p_ref[...], axis=0) + t_ref[...]
  h = jnp.dot(pooled, w1_ref[...], preferred_element_type=jnp.float32)
  h = h + b1_ref[...]
  h = jnp.where(h >= 0.0, h, 0.01 * h)
  o_ref[...] = (
      jnp.dot(h, w2_ref[...], preferred_element_type=jnp.float32)
      + b2_ref[...]
  )


def _mlp(partials, tc_part, W1, b1, W2, b2):
  return pl.pallas_call(
      _mlp_body,
      out_shape=jax.ShapeDtypeStruct((_NUM_GRAPHS, _D_OUT), jnp.float32),
  )(partials, tc_part, W1, b1.reshape(1, _D_HID), W2, b2.reshape(1, _D_OUT))


def kernel(x, edge_index, batch, W1, b1, W2, b2):
  del edge_index
  idx = batch.astype(jnp.int32)
  tc_part = _tc_pool(x[_N_SC:], idx[_N_SC:])
  partials = _seg_pool(x, idx)
  return _mlp(partials, tc_part, W1, b1, W2, b2)
